# baseline (speedup 1.0000x reference)
import functools

import jax
import jax.numpy as jnp
from jax import lax
from jax.experimental import pallas as pl
from jax.experimental.pallas import tpu as pltpu
from jax.experimental.pallas import tpu_sc as plsc

VOCAB = 1000000
DIM = 64
B = 4096
L = 200
N = B * L
NC = 2
NS = 16
NW = NC * NS
N_PER_W = N // NW
CHUNK = 128
NCHUNK = N_PER_W // CHUNK
NBUF = 8

_mesh = plsc.VectorSubcoreMesh(core_axis_name="c", subcore_axis_name="s")


@functools.partial(
    pl.kernel,
    mesh=_mesh,
    out_type=jax.ShapeDtypeStruct((NW, N_PER_W, DIM), jnp.float32),
    scratch_types=[
        pltpu.VMEM((NCHUNK, CHUNK), jnp.int32),
        [pltpu.VMEM((CHUNK, DIM), jnp.float32)] * NBUF,
        [pltpu.SemaphoreType.DMA] * NBUF,
        [pltpu.SemaphoreType.DMA] * NBUF,
    ],
    compiler_params=pltpu.CompilerParams(use_tc_tiling_on_sc=False),
)
def _emb_lookup(tokens_hbm, weight_hbm, out_hbm, idx_v, rows, gsem, ssem):
    wid = lax.axis_index("s") * NC + lax.axis_index("c")
    pltpu.sync_copy(tokens_hbm.at[wid], idx_v)

    def fire_gather(b, c):
        pltpu.async_copy(weight_hbm.at[idx_v.at[c]], rows[b], gsem[b])

    def fire_store(b, c):
        pltpu.async_copy(
            rows[b], out_hbm.at[wid, pl.ds(c * CHUNK, CHUNK)], ssem[b])

    def wait_gather(b):
        pltpu.make_async_copy(weight_hbm.at[idx_v.at[0]], rows[b], gsem[b]).wait()

    def wait_store(b, c):
        pltpu.make_async_copy(
            rows[b], out_hbm.at[wid, pl.ds(c * CHUNK, CHUNK)], ssem[b]).wait()

    for b in range(NBUF):
        fire_gather(b, b)

    @pl.loop(0, NCHUNK, step=NBUF)
    def _(g):
        for b in range(NBUF):
            wait_gather(b)
            fire_store(b, g + b)
        for b in range(NBUF):
            @pl.when(g + b + NBUF < NCHUNK)
            def _():
                wait_store(b, g + b)
                fire_gather(b, g + b + NBUF)

    for b in range(NBUF):
        wait_store(b, NCHUNK - NBUF + b)


def kernel(tokens, weight):
    tokens3d = tokens.reshape(NW, NCHUNK, CHUNK).astype(jnp.int32)
    out = _emb_lookup(tokens3d, weight)
    out2 = lax.optimization_barrier(out.reshape(N // 2, 2 * DIM))
    return out2.reshape(B, L, DIM)
